# baseline (speedup 1.0000x reference)
import jax
import jax.numpy as jnp
from jax import lax
from jax.experimental import pallas as pl
from jax.experimental.pallas import tpu as pltpu
from jax.experimental.pallas import tpu_sc as plsc

N = 10000
E = 160000
D = 128
DE = 16
PK = D // DE
PE = E // PK

NC = 2
NS = 16
K = 128
KP = K // PK
NCH = E // K
NW = NC * NS
CPW = -(-NCH // NW)
NCHP = NW * CPW
RPT = 632
NP = RPT * NS

NB = 4
NB_PAIR = 3
WD = DE + D

_MESH = plsc.VectorSubcoreMesh(core_axis_name="c", subcore_axis_name="s",
                               num_cores=NC, num_subcores=NS)

_SC_PARAMS = pltpu.CompilerParams(use_tc_tiling_on_sc=False)


def _wid():
    return lax.axis_index("s") * NC + lax.axis_index("c")


def _gather_pair_body(tab_hbm, src_hbm, trg_hbm, xs_hbm, xt_hbm,
                      idx_s, idx_t, rows_s, rows_t, gsem_s, gsem_t,
                      wsem_s, wsem_t):
    w = _wid()
    c0 = w * CPW
    nc = jnp.minimum(NCH - c0, CPW)
    pltpu.sync_copy(src_hbm.at[pl.ds(c0, CPW)], idx_s)
    pltpu.sync_copy(trg_hbm.at[pl.ds(c0, CPW)], idx_t)

    def blk_body(blk, carry):
        for b in range(NB_PAIR):
            t = blk * NB_PAIR + b

            @pl.when(t < nc)
            def _():
                base = (c0 + t) * K

                @pl.when(blk > 0)
                def _():
                    pltpu.make_async_copy(
                        rows_s.at[b], xs_hbm.at[pl.ds(base, K)], wsem_s.at[b]
                    ).wait()
                    pltpu.make_async_copy(
                        rows_t.at[b], xt_hbm.at[pl.ds(base, K)], wsem_t.at[b]
                    ).wait()

                pltpu.async_copy(tab_hbm.at[idx_s.at[t]], rows_s.at[b],
                                 gsem_s.at[b])
                pltpu.async_copy(tab_hbm.at[idx_t.at[t]], rows_t.at[b],
                                 gsem_t.at[b])
        for b in range(NB_PAIR):
            t = blk * NB_PAIR + b

            @pl.when(t < nc)
            def _():
                base = (c0 + t) * K
                pltpu.make_async_copy(tab_hbm.at[idx_s.at[t]], rows_s.at[b],
                                      gsem_s.at[b]).wait()
                pltpu.async_copy(rows_s.at[b], xs_hbm.at[pl.ds(base, K)],
                                 wsem_s.at[b])
                pltpu.make_async_copy(tab_hbm.at[idx_t.at[t]], rows_t.at[b],
                                      gsem_t.at[b]).wait()
                pltpu.async_copy(rows_t.at[b], xt_hbm.at[pl.ds(base, K)],
                                 wsem_t.at[b])
        return carry

    lax.fori_loop(0, -(-CPW // NB_PAIR), blk_body, 0)
    for b in range(NB_PAIR):
        @pl.when(b < nc)
        def _():
            pltpu.make_async_copy(rows_s.at[b], xs_hbm.at[pl.ds(0, K)],
                                  wsem_s.at[b]).wait()
            pltpu.make_async_copy(rows_t.at[b], xt_hbm.at[pl.ds(0, K)],
                                  wsem_t.at[b]).wait()


def _gather_pair(tab, src2d, trg2d):
    f = pl.kernel(
        _gather_pair_body,
        out_type=(jax.ShapeDtypeStruct((E, D), jnp.float32),
                  jax.ShapeDtypeStruct((E, D), jnp.float32)),
        mesh=_MESH,
        compiler_params=_SC_PARAMS,
        scratch_types=[
            pltpu.VMEM((CPW, K), jnp.int32),
            pltpu.VMEM((CPW, K), jnp.int32),
            pltpu.VMEM((NB_PAIR, K, D), jnp.float32),
            pltpu.VMEM((NB_PAIR, K, D), jnp.float32),
            pltpu.SemaphoreType.DMA((NB_PAIR,)),
            pltpu.SemaphoreType.DMA((NB_PAIR,)),
            pltpu.SemaphoreType.DMA((NB_PAIR,)),
            pltpu.SemaphoreType.DMA((NB_PAIR,)),
        ],
    )
    return f(tab, src2d, trg2d)


def _gather_one_tab_body(p0_hbm, src_hbm, g0_hbm,
                         idx_s, rows_0, pk_0, gsem_0, wsem_0):
    w = _wid()
    c0 = w * CPW
    nc = jnp.minimum(NCH - c0, CPW)
    pltpu.sync_copy(src_hbm.at[pl.ds(c0, CPW)], idx_s)

    def blk_body(blk, carry):
        for b in range(NB):
            t = blk * NB + b

            @pl.when(t < nc)
            def _():
                basep = (c0 + t) * KP

                @pl.when(blk > 0)
                def _():
                    pltpu.make_async_copy(
                        pk_0.at[b], g0_hbm.at[pl.ds(basep, KP)], wsem_0.at[b]
                    ).wait()

                pltpu.async_copy(p0_hbm.at[idx_s.at[t]], rows_0.at[b],
                                 gsem_0.at[b])
        for b in range(NB):
            t = blk * NB + b

            @pl.when(t < nc)
            def _():
                basep = (c0 + t) * KP
                pltpu.make_async_copy(p0_hbm.at[idx_s.at[t]], rows_0.at[b],
                                      gsem_0.at[b]).wait()
                for r in range(KP):
                    for c in range(PK):
                        pk_0[b, r, pl.ds(DE * c, DE)] = rows_0[b, PK * r + c, :]
                pltpu.async_copy(pk_0.at[b],
                                 g0_hbm.at[pl.ds(basep, KP)], wsem_0.at[b])
        return carry

    lax.fori_loop(0, -(-CPW // NB), blk_body, 0)
    for b in range(NB):
        @pl.when(b < nc)
        def _():
            pltpu.make_async_copy(pk_0.at[b],
                                  g0_hbm.at[pl.ds(0, KP)], wsem_0.at[b]).wait()


def _gather_one_tab(p0, src2d):
    f = pl.kernel(
        _gather_one_tab_body,
        out_type=jax.ShapeDtypeStruct((PE, D), jnp.float32),
        mesh=_MESH,
        compiler_params=_SC_PARAMS,
        scratch_types=[
            pltpu.VMEM((CPW, K), jnp.int32),
            pltpu.VMEM((NB, K, DE), jnp.float32),
            pltpu.VMEM((NB, KP, D), jnp.float32),
            pltpu.SemaphoreType.DMA((NB,)),
            pltpu.SemaphoreType.DMA((NB,)),
        ],
    )
    return f(p0, src2d)


NB_XE = 2
K2 = 64
KP2 = K2 // PK
NCH2 = E // K2
CPW2 = -(-NCH2 // NW)
NCHP2 = NW * CPW2


def _scatter_xe_body(xw_hbm, ef_hbm, src_hbm, trg_hbm, zw_hbm, sp_hbm,
                     idx_s, idx_t, wide, efb, acc, gsem, lsem, ssem):
    cid = lax.axis_index("c")
    sid = lax.axis_index("s")
    row0 = sid * RPT
    pltpu.sync_copy(zw_hbm, acc.at[pl.ds(row0, RPT)])
    plsc.subcore_barrier()

    w = _wid()
    c0 = w * CPW2
    nc = jnp.minimum(NCH2 - c0, CPW2)
    pltpu.sync_copy(src_hbm.at[pl.ds(c0, CPW2)], idx_s)
    pltpu.sync_copy(trg_hbm.at[pl.ds(c0, CPW2)], idx_t)

    def blk_body(blk, carry):
        for b in range(NB_XE):
            t = blk * NB_XE + b

            @pl.when(t < nc)
            def _():
                @pl.when(blk > 0)
                def _():
                    pltpu.make_async_copy(wide.at[b], acc.at[idx_t.at[t]],
                                          ssem.at[b]).wait()

                pltpu.async_copy(xw_hbm.at[idx_s.at[t]], wide.at[b],
                                 gsem.at[b])
                basep = (c0 + t) * KP2
                pltpu.async_copy(ef_hbm.at[pl.ds(basep, KP2)],
                                 efb.at[b], lsem.at[b])
        for b in range(NB_XE):
            t = blk * NB_XE + b

            @pl.when(t < nc)
            def _():
                pltpu.make_async_copy(xw_hbm.at[idx_s.at[t]], wide.at[b],
                                      gsem.at[b]).wait()
                pltpu.make_async_copy(
                    ef_hbm.at[pl.ds((c0 + t) * KP2, KP2)], efb.at[b],
                    lsem.at[b]).wait()
                for r in range(KP2):
                    for c in range(PK):
                        wide[b, PK * r + c, pl.ds(0, DE)] = \
                            efb[b, r, pl.ds(DE * c, DE)]
                pltpu.async_copy(wide.at[b], acc.at[idx_t.at[t]],
                                 ssem.at[b], add=True)
        return carry

    lax.fori_loop(0, -(-CPW2 // NB_XE), blk_body, 0)
    for b in range(NB_XE):
        @pl.when(b < nc)
        def _():
            pltpu.make_async_copy(wide.at[b], acc.at[idx_t.at[b]],
                                  ssem.at[b]).wait()
    plsc.subcore_barrier()
    pltpu.sync_copy(acc.at[pl.ds(row0, RPT)],
                    sp_hbm.at[cid, pl.ds(row0, RPT)])


def _scatter_xe(xw, ef_p, src2d, trg2d, zw):
    f = pl.kernel(
        _scatter_xe_body,
        out_type=jax.ShapeDtypeStruct((NC, NP, WD), jnp.float32),
        mesh=_MESH,
        compiler_params=_SC_PARAMS,
        scratch_types=[
            pltpu.VMEM((CPW2, K2), jnp.int32),
            pltpu.VMEM((CPW2, K2), jnp.int32),
            pltpu.VMEM((NB_XE, K2, WD), jnp.float32),
            pltpu.VMEM((NB_XE, KP2, D), jnp.float32),
            pltpu.VMEM_SHARED((NP, WD), jnp.float32),
            pltpu.SemaphoreType.DMA((NB_XE,)),
            pltpu.SemaphoreType.DMA((NB_XE,)),
            pltpu.SemaphoreType.DMA((NB_XE,)),
        ],
    )
    return f(xw, ef_p, src2d, trg2d, zw)


def _scatter_e_body(ef_hbm, trg_hbm, ze_hbm, sep_hbm,
                    idx_t, efb, efb2, acce, gsem, ssem):
    cid = lax.axis_index("c")
    sid = lax.axis_index("s")
    row0 = sid * RPT
    pltpu.sync_copy(ze_hbm, acce.at[pl.ds(row0, RPT)])
    plsc.subcore_barrier()

    w = _wid()
    c0 = w * CPW
    nc = jnp.minimum(NCH - c0, CPW)
    pltpu.sync_copy(trg_hbm.at[pl.ds(c0, CPW)], idx_t)

    def blk_body(blk, carry):
        for b in range(NB):
            t = blk * NB + b

            @pl.when(t < nc)
            def _():
                basep = (c0 + t) * KP

                @pl.when(blk > 0)
                def _():
                    pltpu.make_async_copy(efb2.at[b],
                                          acce.at[idx_t.at[t]],
                                          ssem.at[b]).wait()

                pltpu.async_copy(ef_hbm.at[pl.ds(basep, KP)],
                                 efb.at[b], gsem.at[b])
        for b in range(NB):
            t = blk * NB + b

            @pl.when(t < nc)
            def _():
                basep = (c0 + t) * KP
                pltpu.make_async_copy(ef_hbm.at[pl.ds(basep, KP)],
                                      efb.at[b], gsem.at[b]).wait()
                for r in range(KP):
                    for c in range(PK):
                        efb2[b, PK * r + c, :] = efb[b, r, pl.ds(DE * c, DE)]
                pltpu.async_copy(efb2.at[b],
                                 acce.at[idx_t.at[t]],
                                 ssem.at[b], add=True)
        return carry

    lax.fori_loop(0, -(-CPW // NB), blk_body, 0)
    for b in range(NB):
        @pl.when(b < nc)
        def _():
            pltpu.make_async_copy(efb2.at[b],
                                  acce.at[idx_t.at[b]], ssem.at[b]).wait()
    plsc.subcore_barrier()
    pltpu.sync_copy(acce.at[pl.ds(row0, RPT)],
                    sep_hbm.at[cid, pl.ds(row0, RPT)])


def _scatter_e(ef_p, trg2d, ze):
    f = pl.kernel(
        _scatter_e_body,
        out_type=jax.ShapeDtypeStruct((NC, NP, DE), jnp.float32),
        mesh=_MESH,
        compiler_params=_SC_PARAMS,
        scratch_types=[
            pltpu.VMEM((CPW, K), jnp.int32),
            pltpu.VMEM((NB, KP, D), jnp.float32),
            pltpu.VMEM((NB, K, DE), jnp.float32),
            pltpu.VMEM_SHARED((NP, DE), jnp.float32),
            pltpu.SemaphoreType.DMA((NB,)),
            pltpu.SemaphoreType.DMA((NB,)),
        ],
    )
    return f(ef_p, trg2d, ze)


NBLK = 2000


def _node_update_body(sp0, sp1, x_ref, w2, wself,
                      xn_ref, xr_ref, xw_ref):
    sp = sp0[...] + sp1[...]
    xn = jnp.dot(sp, w2[...], preferred_element_type=jnp.float32)
    xn += jnp.dot(x_ref[...], wself[...], preferred_element_type=jnp.float32)
    xn_ref[...] = xn
    xr = jnp.maximum(xn, 0.0)
    xr_ref[...] = xr
    xw_ref[...] = jnp.concatenate(
        [jnp.zeros((NBLK, DE), jnp.float32), xr], axis=1)


def _node_update(sp0, sp1, x, w2, wself):
    grid = N // NBLK
    return pl.pallas_call(
        _node_update_body,
        grid=(grid,),
        in_specs=[
            pl.BlockSpec((NBLK, WD), lambda i: (i, 0)),
            pl.BlockSpec((NBLK, WD), lambda i: (i, 0)),
            pl.BlockSpec((NBLK, D), lambda i: (i, 0)),
            pl.BlockSpec((WD, D), lambda i: (0, 0)),
            pl.BlockSpec((D, D), lambda i: (0, 0)),
        ],
        out_specs=(pl.BlockSpec((NBLK, D), lambda i: (i, 0)),
                   pl.BlockSpec((NBLK, D), lambda i: (i, 0)),
                   pl.BlockSpec((NBLK, WD), lambda i: (i, 0))),
        out_shape=(jax.ShapeDtypeStruct((N, D), jnp.float32),
                   jax.ShapeDtypeStruct((N, D), jnp.float32),
                   jax.ShapeDtypeStruct((N, WD), jnp.float32)),
    )(sp0, sp1, x, w2, wself)


def _widen_body(x_ref, xw_ref):
    xw_ref[...] = jnp.concatenate(
        [jnp.zeros((NBLK, DE), jnp.float32), x_ref[...]], axis=1)


def _widen(x):
    grid = N // NBLK
    return pl.pallas_call(
        _widen_body,
        grid=(grid,),
        in_specs=[pl.BlockSpec((NBLK, D), lambda i: (i, 0))],
        out_specs=pl.BlockSpec((NBLK, WD), lambda i: (i, 0)),
        out_shape=jax.ShapeDtypeStruct((N, WD), jnp.float32),
    )(x)


EBLK = 3200
PBLK = EBLK // PK


def _edge_update_body(efp_ref, xs_ref, xt_ref, A8_ref, B_ref, C_ref, Dm_ref,
                      w_ref, b128_ref, out_ref):
    xs = xs_ref[...]
    xt = xt_ref[...]
    dot_st = jnp.sum(xs * xt, axis=1)
    na = jnp.sqrt(jnp.sum(xs * xs, axis=1))
    nb = jnp.sqrt(jnp.sum(xt * xt, axis=1))
    sim = dot_st / (jnp.maximum(na, 1e-8) * jnp.maximum(nb, 1e-8))
    o = jnp.dot(xs, B_ref[...], preferred_element_type=jnp.float32)
    o += jnp.dot(xt, C_ref[...], preferred_element_type=jnp.float32)
    o += jnp.dot(jnp.abs(xs - xt), Dm_ref[...],
                 preferred_element_type=jnp.float32)
    o += sim[:, None] * w_ref[...]
    o3 = o.reshape(PBLK, PK, DE)
    op = jnp.concatenate([o3[:, g, :] for g in range(PK)], axis=1)
    op += jnp.dot(efp_ref[...], A8_ref[...],
                  preferred_element_type=jnp.float32)
    out_ref[...] = op + b128_ref[...]


def _edge_update(ef_p, xs, xt, A8, B, C, Dm, w, b128):
    grid = E // EBLK
    return pl.pallas_call(
        _edge_update_body,
        grid=(grid,),
        in_specs=[
            pl.BlockSpec((PBLK, D), lambda i: (i, 0)),
            pl.BlockSpec((EBLK, D), lambda i: (i, 0)),
            pl.BlockSpec((EBLK, D), lambda i: (i, 0)),
            pl.BlockSpec((D, D), lambda i: (0, 0)),
            pl.BlockSpec((D, DE), lambda i: (0, 0)),
            pl.BlockSpec((D, DE), lambda i: (0, 0)),
            pl.BlockSpec((D, DE), lambda i: (0, 0)),
            pl.BlockSpec((1, DE), lambda i: (0, 0)),
            pl.BlockSpec((1, D), lambda i: (0, 0)),
        ],
        out_specs=pl.BlockSpec((PBLK, D), lambda i: (i, 0)),
        out_shape=jax.ShapeDtypeStruct((PE, D), jnp.float32),
    )(ef_p, xs, xt, A8, B, C, Dm, w, b128)


def _tab_combine_body(t0, t1, we, out_ref):
    out_ref[...] = jnp.dot(t0[...] + t1[...], we[...],
                           preferred_element_type=jnp.float32)


def _tab_combine(t0, t1, We):
    return pl.pallas_call(
        _tab_combine_body,
        grid=(2,),
        in_specs=[
            pl.BlockSpec((NP // 2, DE), lambda i: (i, 0)),
            pl.BlockSpec((NP // 2, DE), lambda i: (i, 0)),
            pl.BlockSpec((DE, DE), lambda i: (0, 0)),
        ],
        out_specs=pl.BlockSpec((NP // 2, DE), lambda i: (i, 0)),
        out_shape=jax.ShapeDtypeStruct((NP, DE), jnp.float32),
    )(t0, t1, We)


LBLK = 2000


def _line_combine_body(g0, efm, wes8, out_ref):
    out = g0[...]
    out += jnp.dot(efm[...], wes8[...], preferred_element_type=jnp.float32)
    out_ref[...] = out


def _line_combine(g0_p, efm_p, Wes8):
    grid = PE // LBLK
    return pl.pallas_call(
        _line_combine_body,
        grid=(grid,),
        in_specs=[
            pl.BlockSpec((LBLK, D), lambda i: (i, 0)),
            pl.BlockSpec((LBLK, D), lambda i: (i, 0)),
            pl.BlockSpec((D, D), lambda i: (0, 0)),
        ],
        out_specs=pl.BlockSpec((LBLK, D), lambda i: (i, 0)),
        out_shape=jax.ShapeDtypeStruct((PE, D), jnp.float32),
    )(g0_p, efm_p, Wes8)


def kernel(x, edge_index, edge_feat, line_edge_index, W_msg0, W_self0,
           W_fc0, b_fc0, W_e0, W_eself0, W_msg1, W_self1, W_fc1, b_fc1,
           W_e1, W_eself1):
    pad = ((0, NCHP - NCH), (0, 0))
    src2d = jnp.pad(edge_index[0].reshape(NCH, K), pad)
    trg2d = jnp.pad(edge_index[1].reshape(NCH, K), pad)
    pad2 = ((0, NCHP2 - NCH2), (0, 0))
    src2d64 = jnp.pad(edge_index[0].reshape(NCH2, K2), pad2)
    trg2d64 = jnp.pad(edge_index[1].reshape(NCH2, K2), pad2)
    zw = jnp.zeros((RPT, WD), jnp.float32)
    ze = jnp.zeros((RPT, DE), jnp.float32)
    eye8 = jnp.eye(PK, dtype=jnp.float32)

    ef_p = edge_feat.reshape(PE, D)
    params = [(W_msg0, W_self0, W_fc0, b_fc0, W_e0, W_eself0),
              (W_msg1, W_self1, W_fc1, b_fc1, W_e1, W_eself1)]
    xw = _widen(x)
    for li, (W_msg, W_self, W_fc, b_fc, W_e, W_eself) in enumerate(params):
        A8 = jnp.kron(eye8, W_fc[:DE])
        B = W_fc[DE:DE + D]
        C = W_fc[DE + D:DE + 2 * D]
        Dm = W_fc[DE + 2 * D:DE + 3 * D]
        wrow = W_fc[DE + 3 * D:DE + 3 * D + 1]
        b128 = jnp.tile(b_fc, PK)[None, :]
        Wes8 = jnp.kron(eye8, W_eself)
        W2 = jnp.concatenate([W_msg[D:], W_msg[:D]], axis=0)

        sp = _scatter_xe(xw, ef_p, src2d64, trg2d64, zw)
        xn, xr, xw = _node_update(sp[0, :N], sp[1, :N], x, W2, W_self)
        xs, xt = _gather_pair(xn, src2d, trg2d)
        efm_p = _edge_update(ef_p, xs, xt, A8, B, C, Dm, wrow, b128)
        tmp = _scatter_e(efm_p, trg2d, ze)
        T = _tab_combine(tmp[0], tmp[1], W_e)
        g_p = _gather_one_tab(T, src2d)
        ef_p = _line_combine(g_p, efm_p, Wes8)
        x = xr
    return ef_p.reshape(E, DE)


# speedup vs baseline: 1.1112x; 1.1112x over previous
import jax
import jax.numpy as jnp
from jax import lax
from jax.experimental import pallas as pl
from jax.experimental.pallas import tpu as pltpu
from jax.experimental.pallas import tpu_sc as plsc

N = 10000
E = 160000
D = 128
DE = 16
PK = D // DE
PE = E // PK

NC = 2
NS = 16
K = 128
KP = K // PK
NCH = E // K
NW = NC * NS
CPW = -(-NCH // NW)
NCHP = NW * CPW
RPT = 632
NP = RPT * NS

NB = 4
NB_PAIR = 3
NB_SCX = 2

_MESH = plsc.VectorSubcoreMesh(core_axis_name="c", subcore_axis_name="s",
                               num_cores=NC, num_subcores=NS)

_SC_PARAMS = pltpu.CompilerParams(use_tc_tiling_on_sc=False)


def _wid():
    return lax.axis_index("s") * NC + lax.axis_index("c")


def _gather_pair_body(tab_hbm, src_hbm, trg_hbm, xs_hbm, xt_hbm,
                      idx_s, idx_t, rows_s, rows_t, gsem_s, gsem_t,
                      wsem_s, wsem_t):
    w = _wid()
    c0 = w * CPW
    nc = jnp.minimum(NCH - c0, CPW)
    pltpu.sync_copy(src_hbm.at[pl.ds(c0, CPW)], idx_s)
    pltpu.sync_copy(trg_hbm.at[pl.ds(c0, CPW)], idx_t)

    def blk_body(blk, carry):
        for b in range(NB_PAIR):
            t = blk * NB_PAIR + b

            @pl.when(t < nc)
            def _():
                base = (c0 + t) * K

                @pl.when(blk > 0)
                def _():
                    pltpu.make_async_copy(
                        rows_s.at[b], xs_hbm.at[pl.ds(base, K)], wsem_s.at[b]
                    ).wait()
                    pltpu.make_async_copy(
                        rows_t.at[b], xt_hbm.at[pl.ds(base, K)], wsem_t.at[b]
                    ).wait()

                pltpu.async_copy(tab_hbm.at[idx_s.at[t]], rows_s.at[b],
                                 gsem_s.at[b])
                pltpu.async_copy(tab_hbm.at[idx_t.at[t]], rows_t.at[b],
                                 gsem_t.at[b])
        for b in range(NB_PAIR):
            t = blk * NB_PAIR + b

            @pl.when(t < nc)
            def _():
                base = (c0 + t) * K
                pltpu.make_async_copy(tab_hbm.at[idx_s.at[t]], rows_s.at[b],
                                      gsem_s.at[b]).wait()
                pltpu.async_copy(rows_s.at[b], xs_hbm.at[pl.ds(base, K)],
                                 wsem_s.at[b])
                pltpu.make_async_copy(tab_hbm.at[idx_t.at[t]], rows_t.at[b],
                                      gsem_t.at[b]).wait()
                pltpu.async_copy(rows_t.at[b], xt_hbm.at[pl.ds(base, K)],
                                 wsem_t.at[b])
        return carry

    lax.fori_loop(0, -(-CPW // NB_PAIR), blk_body, 0)
    for b in range(NB_PAIR):
        @pl.when(b < nc)
        def _():
            pltpu.make_async_copy(rows_s.at[b], xs_hbm.at[pl.ds(0, K)],
                                  wsem_s.at[b]).wait()
            pltpu.make_async_copy(rows_t.at[b], xt_hbm.at[pl.ds(0, K)],
                                  wsem_t.at[b]).wait()


def _gather_pair(tab, src2d, trg2d):
    f = pl.kernel(
        _gather_pair_body,
        out_type=(jax.ShapeDtypeStruct((E, D), jnp.float32),
                  jax.ShapeDtypeStruct((E, D), jnp.float32)),
        mesh=_MESH,
        compiler_params=_SC_PARAMS,
        scratch_types=[
            pltpu.VMEM((CPW, K), jnp.int32),
            pltpu.VMEM((CPW, K), jnp.int32),
            pltpu.VMEM((NB_PAIR, K, D), jnp.float32),
            pltpu.VMEM((NB_PAIR, K, D), jnp.float32),
            pltpu.SemaphoreType.DMA((NB_PAIR,)),
            pltpu.SemaphoreType.DMA((NB_PAIR,)),
            pltpu.SemaphoreType.DMA((NB_PAIR,)),
            pltpu.SemaphoreType.DMA((NB_PAIR,)),
        ],
    )
    return f(tab, src2d, trg2d)


NB_G = 8


def _gather_one_tab_body(p0_hbm, src_hbm, g0_hbm,
                         idx_s, rows_0, pk_0, gsem_0, wsem_0):
    w = _wid()
    c0 = w * CPW
    nc = jnp.minimum(NCH - c0, CPW)
    pltpu.sync_copy(src_hbm.at[pl.ds(c0, CPW)], idx_s)

    def blk_body(blk, carry):
        for b in range(NB_G):
            t = blk * NB_G + b

            @pl.when(t < nc)
            def _():
                basep = (c0 + t) * KP

                @pl.when(blk > 0)
                def _():
                    pltpu.make_async_copy(
                        pk_0.at[b], g0_hbm.at[pl.ds(basep, KP)], wsem_0.at[b]
                    ).wait()

                pltpu.async_copy(p0_hbm.at[idx_s.at[t]], rows_0.at[b],
                                 gsem_0.at[b])
        for b in range(NB_G):
            t = blk * NB_G + b

            @pl.when(t < nc)
            def _():
                basep = (c0 + t) * KP
                pltpu.make_async_copy(p0_hbm.at[idx_s.at[t]], rows_0.at[b],
                                      gsem_0.at[b]).wait()
                for r in range(KP):
                    for c in range(PK):
                        pk_0[b, r, pl.ds(DE * c, DE)] = rows_0[b, PK * r + c, :]
                pltpu.async_copy(pk_0.at[b],
                                 g0_hbm.at[pl.ds(basep, KP)], wsem_0.at[b])
        return carry

    lax.fori_loop(0, -(-CPW // NB_G), blk_body, 0)
    for b in range(NB_G):
        @pl.when(b < nc)
        def _():
            pltpu.make_async_copy(pk_0.at[b],
                                  g0_hbm.at[pl.ds(0, KP)], wsem_0.at[b]).wait()


def _gather_one_tab(p0, src2d):
    f = pl.kernel(
        _gather_one_tab_body,
        out_type=jax.ShapeDtypeStruct((PE, D), jnp.float32),
        mesh=_MESH,
        compiler_params=_SC_PARAMS,
        scratch_types=[
            pltpu.VMEM((CPW, K), jnp.int32),
            pltpu.VMEM((NB_G, K, DE), jnp.float32),
            pltpu.VMEM((NB_G, KP, D), jnp.float32),
            pltpu.SemaphoreType.DMA((NB_G,)),
            pltpu.SemaphoreType.DMA((NB_G,)),
        ],
    )
    return f(p0, src2d)


def _scatter_x_body(x_hbm, src_hbm, trg_hbm, zx_hbm, sxp_hbm,
                    idx_s, idx_t, rows, accx, gsem, ssem):
    cid = lax.axis_index("c")
    sid = lax.axis_index("s")
    row0 = sid * RPT
    pltpu.sync_copy(zx_hbm, accx.at[pl.ds(row0, RPT)])
    plsc.subcore_barrier()

    w = _wid()
    c0 = w * CPW
    nc = jnp.minimum(NCH - c0, CPW)
    pltpu.sync_copy(src_hbm.at[pl.ds(c0, CPW)], idx_s)
    pltpu.sync_copy(trg_hbm.at[pl.ds(c0, CPW)], idx_t)

    def blk_body(blk, carry):
        for b in range(NB_SCX):
            t = blk * NB_SCX + b

            @pl.when(t < nc)
            def _():
                @pl.when(blk > 0)
                def _():
                    pltpu.make_async_copy(rows.at[b], accx.at[idx_t.at[t]],
                                          ssem.at[b]).wait()

                pltpu.async_copy(x_hbm.at[idx_s.at[t]], rows.at[b],
                                 gsem.at[b])
        for b in range(NB_SCX):
            t = blk * NB_SCX + b

            @pl.when(t < nc)
            def _():
                pltpu.make_async_copy(x_hbm.at[idx_s.at[t]], rows.at[b],
                                      gsem.at[b]).wait()
                pltpu.async_copy(rows.at[b], accx.at[idx_t.at[t]],
                                 ssem.at[b], add=True)
        return carry

    lax.fori_loop(0, -(-CPW // NB_SCX), blk_body, 0)
    for b in range(NB_SCX):
        @pl.when(b < nc)
        def _():
            pltpu.make_async_copy(rows.at[b], accx.at[idx_t.at[b]],
                                  ssem.at[b]).wait()
    plsc.subcore_barrier()
    pltpu.sync_copy(accx.at[pl.ds(row0, RPT)],
                    sxp_hbm.at[cid, pl.ds(row0, RPT)])


def _scatter_x(x, src2d, trg2d, zx):
    f = pl.kernel(
        _scatter_x_body,
        out_type=jax.ShapeDtypeStruct((NC, NP, D), jnp.float32),
        mesh=_MESH,
        compiler_params=_SC_PARAMS,
        scratch_types=[
            pltpu.VMEM((CPW, K), jnp.int32),
            pltpu.VMEM((CPW, K), jnp.int32),
            pltpu.VMEM((NB_SCX, K, D), jnp.float32),
            pltpu.VMEM_SHARED((NP, D), jnp.float32),
            pltpu.SemaphoreType.DMA((NB_SCX,)),
            pltpu.SemaphoreType.DMA((NB_SCX,)),
        ],
    )
    return f(x, src2d, trg2d, zx)


NB_E = 8


def _scatter_e_body(ef_hbm, trg_hbm, ze_hbm, sep_hbm,
                    idx_t, efb, efb2, acce, gsem, ssem):
    cid = lax.axis_index("c")
    sid = lax.axis_index("s")
    row0 = sid * RPT
    pltpu.sync_copy(ze_hbm, acce.at[pl.ds(row0, RPT)])
    plsc.subcore_barrier()

    w = _wid()
    c0 = w * CPW
    nc = jnp.minimum(NCH - c0, CPW)
    pltpu.sync_copy(trg_hbm.at[pl.ds(c0, CPW)], idx_t)

    def blk_body(blk, carry):
        for b in range(NB_E):
            t = blk * NB_E + b

            @pl.when(t < nc)
            def _():
                basep = (c0 + t) * KP

                @pl.when(blk > 0)
                def _():
                    pltpu.make_async_copy(efb2.at[b],
                                          acce.at[idx_t.at[t]],
                                          ssem.at[b]).wait()

                pltpu.async_copy(ef_hbm.at[pl.ds(basep, KP)],
                                 efb.at[b], gsem.at[b])
        for b in range(NB_E):
            t = blk * NB_E + b

            @pl.when(t < nc)
            def _():
                basep = (c0 + t) * KP
                pltpu.make_async_copy(ef_hbm.at[pl.ds(basep, KP)],
                                      efb.at[b], gsem.at[b]).wait()
                for r in range(KP):
                    for c in range(PK):
                        efb2[b, PK * r + c, :] = efb[b, r, pl.ds(DE * c, DE)]
                pltpu.async_copy(efb2.at[b],
                                 acce.at[idx_t.at[t]],
                                 ssem.at[b], add=True)
        return carry

    lax.fori_loop(0, -(-CPW // NB_E), blk_body, 0)
    for b in range(NB_E):
        @pl.when(b < nc)
        def _():
            pltpu.make_async_copy(efb2.at[b],
                                  acce.at[idx_t.at[b]], ssem.at[b]).wait()
    plsc.subcore_barrier()
    pltpu.sync_copy(acce.at[pl.ds(row0, RPT)],
                    sep_hbm.at[cid, pl.ds(row0, RPT)])


def _scatter_e(ef_p, trg2d, ze):
    f = pl.kernel(
        _scatter_e_body,
        out_type=jax.ShapeDtypeStruct((NC, NP, DE), jnp.float32),
        mesh=_MESH,
        compiler_params=_SC_PARAMS,
        scratch_types=[
            pltpu.VMEM((CPW, K), jnp.int32),
            pltpu.VMEM((NB_E, KP, D), jnp.float32),
            pltpu.VMEM((NB_E, K, DE), jnp.float32),
            pltpu.VMEM_SHARED((NP, DE), jnp.float32),
            pltpu.SemaphoreType.DMA((NB_E,)),
            pltpu.SemaphoreType.DMA((NB_E,)),
        ],
    )
    return f(ef_p, trg2d, ze)


NBLK = 2000


def _node_update_body(sx0, sx1, se0, se1, x_ref, wm1, wm2, wself,
                      xn_ref, xr_ref):
    sx = sx0[...] + sx1[...]
    se = se0[...] + se1[...]
    xn = jnp.dot(sx, wm1[...], preferred_element_type=jnp.float32)
    xn += jnp.dot(se, wm2[...], preferred_element_type=jnp.float32)
    xn += jnp.dot(x_ref[...], wself[...], preferred_element_type=jnp.float32)
    xn_ref[...] = xn
    xr_ref[...] = jnp.maximum(xn, 0.0)


def _node_update(sx0, sx1, se0, se1, x, wm1, wm2, wself):
    grid = N // NBLK
    return pl.pallas_call(
        _node_update_body,
        grid=(grid,),
        in_specs=[
            pl.BlockSpec((NBLK, D), lambda i: (i, 0)),
            pl.BlockSpec((NBLK, D), lambda i: (i, 0)),
            pl.BlockSpec((NBLK, DE), lambda i: (i, 0)),
            pl.BlockSpec((NBLK, DE), lambda i: (i, 0)),
            pl.BlockSpec((NBLK, D), lambda i: (i, 0)),
            pl.BlockSpec((D, D), lambda i: (0, 0)),
            pl.BlockSpec((DE, D), lambda i: (0, 0)),
            pl.BlockSpec((D, D), lambda i: (0, 0)),
        ],
        out_specs=(pl.BlockSpec((NBLK, D), lambda i: (i, 0)),
                   pl.BlockSpec((NBLK, D), lambda i: (i, 0))),
        out_shape=(jax.ShapeDtypeStruct((N, D), jnp.float32),
                   jax.ShapeDtypeStruct((N, D), jnp.float32)),
    )(sx0, sx1, se0, se1, x, wm1, wm2, wself)


EBLK = 3200
PBLK = EBLK // PK


def _edge_update_body(efp_ref, xs_ref, xt_ref, A8_ref, B_ref, C_ref, Dm_ref,
                      w_ref, b128_ref, out_ref):
    xs = xs_ref[...]
    xt = xt_ref[...]
    dot_st = jnp.sum(xs * xt, axis=1)
    na = jnp.sqrt(jnp.sum(xs * xs, axis=1))
    nb = jnp.sqrt(jnp.sum(xt * xt, axis=1))
    sim = dot_st / (jnp.maximum(na, 1e-8) * jnp.maximum(nb, 1e-8))
    o = jnp.dot(xs, B_ref[...], preferred_element_type=jnp.float32)
    o += jnp.dot(xt, C_ref[...], preferred_element_type=jnp.float32)
    o += jnp.dot(jnp.abs(xs - xt), Dm_ref[...],
                 preferred_element_type=jnp.float32)
    o += sim[:, None] * w_ref[...]
    o3 = o.reshape(PBLK, PK, DE)
    op = jnp.concatenate([o3[:, g, :] for g in range(PK)], axis=1)
    op += jnp.dot(efp_ref[...], A8_ref[...],
                  preferred_element_type=jnp.float32)
    out_ref[...] = op + b128_ref[...]


def _edge_update(ef_p, xs, xt, A8, B, C, Dm, w, b128):
    grid = E // EBLK
    return pl.pallas_call(
        _edge_update_body,
        grid=(grid,),
        in_specs=[
            pl.BlockSpec((PBLK, D), lambda i: (i, 0)),
            pl.BlockSpec((EBLK, D), lambda i: (i, 0)),
            pl.BlockSpec((EBLK, D), lambda i: (i, 0)),
            pl.BlockSpec((D, D), lambda i: (0, 0)),
            pl.BlockSpec((D, DE), lambda i: (0, 0)),
            pl.BlockSpec((D, DE), lambda i: (0, 0)),
            pl.BlockSpec((D, DE), lambda i: (0, 0)),
            pl.BlockSpec((1, DE), lambda i: (0, 0)),
            pl.BlockSpec((1, D), lambda i: (0, 0)),
        ],
        out_specs=pl.BlockSpec((PBLK, D), lambda i: (i, 0)),
        out_shape=jax.ShapeDtypeStruct((PE, D), jnp.float32),
    )(ef_p, xs, xt, A8, B, C, Dm, w, b128)


def _tab_combine_body(t0, t1, we, out_ref):
    out_ref[...] = jnp.dot(t0[...] + t1[...], we[...],
                           preferred_element_type=jnp.float32)


def _tab_combine(t0, t1, We):
    return pl.pallas_call(
        _tab_combine_body,
        grid=(2,),
        in_specs=[
            pl.BlockSpec((NP // 2, DE), lambda i: (i, 0)),
            pl.BlockSpec((NP // 2, DE), lambda i: (i, 0)),
            pl.BlockSpec((DE, DE), lambda i: (0, 0)),
        ],
        out_specs=pl.BlockSpec((NP // 2, DE), lambda i: (i, 0)),
        out_shape=jax.ShapeDtypeStruct((NP, DE), jnp.float32),
    )(t0, t1, We)


LBLK = 2000


def _line_combine_body(g0, efm, wes8, out_ref):
    out = g0[...]
    out += jnp.dot(efm[...], wes8[...], preferred_element_type=jnp.float32)
    out_ref[...] = out


def _line_combine(g0_p, efm_p, Wes8):
    grid = PE // LBLK
    return pl.pallas_call(
        _line_combine_body,
        grid=(grid,),
        in_specs=[
            pl.BlockSpec((LBLK, D), lambda i: (i, 0)),
            pl.BlockSpec((LBLK, D), lambda i: (i, 0)),
            pl.BlockSpec((D, D), lambda i: (0, 0)),
        ],
        out_specs=pl.BlockSpec((LBLK, D), lambda i: (i, 0)),
        out_shape=jax.ShapeDtypeStruct((PE, D), jnp.float32),
    )(g0_p, efm_p, Wes8)


def kernel(x, edge_index, edge_feat, line_edge_index, W_msg0, W_self0,
           W_fc0, b_fc0, W_e0, W_eself0, W_msg1, W_self1, W_fc1, b_fc1,
           W_e1, W_eself1):
    pad = ((0, NCHP - NCH), (0, 0))
    src2d = jnp.pad(edge_index[0].reshape(NCH, K), pad)
    trg2d = jnp.pad(edge_index[1].reshape(NCH, K), pad)
    zx = jnp.zeros((RPT, D), jnp.float32)
    ze = jnp.zeros((RPT, DE), jnp.float32)
    eye8 = jnp.eye(PK, dtype=jnp.float32)

    ef_p = edge_feat.reshape(PE, D)
    params = [(W_msg0, W_self0, W_fc0, b_fc0, W_e0, W_eself0),
              (W_msg1, W_self1, W_fc1, b_fc1, W_e1, W_eself1)]
    sxp = _scatter_x(x, src2d, trg2d, zx)
    for li, (W_msg, W_self, W_fc, b_fc, W_e, W_eself) in enumerate(params):
        A8 = jnp.kron(eye8, W_fc[:DE])
        B = W_fc[DE:DE + D]
        C = W_fc[DE + D:DE + 2 * D]
        Dm = W_fc[DE + 2 * D:DE + 3 * D]
        wrow = W_fc[DE + 3 * D:DE + 3 * D + 1]
        b128 = jnp.tile(b_fc, PK)[None, :]
        Wes8 = jnp.kron(eye8, W_eself)

        sep = _scatter_e(ef_p, trg2d, ze)
        xn, xr = _node_update(sxp[0, :N], sxp[1, :N], sep[0, :N], sep[1, :N],
                              x, W_msg[:D], W_msg[D:], W_self)
        xs, xt = _gather_pair(xn, src2d, trg2d)
        if li == 0:
            sxp = _scatter_x(xr, src2d, trg2d, zx)
        efm_p = _edge_update(ef_p, xs, xt, A8, B, C, Dm, wrow, b128)
        tmp = _scatter_e(efm_p, trg2d, ze)
        T = _tab_combine(tmp[0], tmp[1], W_e)
        g_p = _gather_one_tab(T, src2d)
        ef_p = _line_combine(g_p, efm_p, Wes8)
        x = xr
    return ef_p.reshape(E, DE)
